# Initial kernel scaffold; baseline (speedup 1.0000x reference)
import jax
import jax.numpy as jnp
from jax.experimental import pallas as pl


def kernel(x, edge_index, W1, a_src1, a_dst1, b1, W2, a_src2, a_dst2, b2, W3, a_src3, a_dst3, b3, lin_w, lin_b):
    raise NotImplementedError("write your pallas kernel here")


# speedup vs baseline: 47.5479x; 47.5479x over previous
import functools

import jax
import jax.numpy as jnp
from jax import lax
from jax.experimental import pallas as pl
from jax.experimental.pallas import tpu as pltpu
from jax.experimental.pallas import tpu_sc as plsc

N = 10000
E = 640000
F_IN = 128
H = 4
C = 64

ZW = 128
SW = 16
NTILE = 16
ETOT = E + N
EW = ((ETOT + NTILE * 128 - 1) // (NTILE * 128)) * 128
EPAD = EW * NTILE
CHUNK = 128
NCHUNK = EW // CHUNK
ACC_R = 10240
ROWS_PER_TILE = ACC_R // NTILE
ALW = 16

BLK = 1000
NBLK = N // BLK


def _leaky_exp(e):
    return jnp.exp(jnp.where(e > 0, e, e * jnp.float32(0.2)))


def _sc_body(z_flat, al16f_hbm, srcp, dstp, out_z, out_sf,
             rows, zflat, sidx, gidx, didx,
             ias0, ias1, iad0, iad1, as0b, as1b, ad0b, ad1b,
             w0b, w1b, iw0, iw1, alspf, acc, saccf, sem):
    c = lax.axis_index("c")
    s = lax.axis_index("s")
    zf = jnp.zeros((16,), jnp.float32)
    SSLICE = ACC_R * ALW // NTILE

    pltpu.sync_copy(al16f_hbm.at[pl.ds(s * SSLICE, SSLICE)],
                    alspf.at[pl.ds(s * SSLICE, SSLICE)])

    def zrow(i, _):
        for k in range(ZW // 16):
            rows[i, pl.ds(k * 16, 16)] = zf
        return 0
    lax.fori_loop(0, CHUNK, zrow, 0)

    def zflat_body(i, _):
        zflat[pl.ds(i * 16, 16)] = zf
        return 0
    lax.fori_loop(0, ZFLAT // 16, zflat_body, 0)
    for k in range(ROWS_PER_TILE // CHUNK):
        pltpu.sync_copy(rows, acc.at[pl.ds(s * ROWS_PER_TILE + k * CHUNK, CHUNK)])
    for k in range(SSLICE // ZFLAT):
        pltpu.sync_copy(zflat, saccf.at[pl.ds(s * SSLICE + k * ZFLAT, ZFLAT)])
    plsc.subcore_barrier()

    base_e = s * EW
    c_off = c * N
    cs0 = 2 * c
    cs1 = 2 * c + 1
    cd0 = 4 + 2 * c
    cd1 = 5 + 2 * c

    def chunk_body(i, _):
        eb = base_e + i * CHUNK
        pltpu.sync_copy(srcp.at[pl.ds(eb, CHUNK)], sidx)
        pltpu.sync_copy(dstp.at[pl.ds(eb, CHUNK)], didx)
        for j in range(CHUNK // 16):
            dj = pl.ds(j * 16, 16)
            sv = sidx[dj]
            dv = didx[dj]
            sv16 = sv * ALW
            dv16 = dv * ALW
            gidx[dj] = sv + c_off
            ias0[dj] = sv16 + cs0
            ias1[dj] = sv16 + cs1
            iad0[dj] = dv16 + cd0
            iad1[dj] = dv16 + cd1
            iw0[dj] = dv16
            iw1[dj] = dv16 + 1
        pltpu.async_copy(z_flat.at[gidx], rows, sem).wait()
        pltpu.async_copy(alspf.at[ias0], as0b, sem).wait()
        pltpu.async_copy(alspf.at[ias1], as1b, sem).wait()
        pltpu.async_copy(alspf.at[iad0], ad0b, sem).wait()
        pltpu.async_copy(alspf.at[iad1], ad1b, sem).wait()

        def grp(j, _):
            dj = pl.ds(j * 16, 16)
            w0 = _leaky_exp(as0b[dj] + ad0b[dj])
            w1 = _leaky_exp(as1b[dj] + ad1b[dj])
            w0b[dj] = w0
            w1b[dj] = w1
            for t in range(16):
                eidx = j * 16 + t
                ev = jnp.full((16,), eidx, jnp.int32)
                w0s = plsc.load_gather(w0b, [ev])
                w1s = plsc.load_gather(w1b, [ev])
                for k in range(4):
                    rows[eidx, pl.ds(k * 16, 16)] = rows[eidx, pl.ds(k * 16, 16)] * w0s
                for k in range(4, 8):
                    rows[eidx, pl.ds(k * 16, 16)] = rows[eidx, pl.ds(k * 16, 16)] * w1s
            return 0
        lax.fori_loop(0, CHUNK // 16, grp, 0)
        pltpu.sync_copy(rows, acc.at[didx], add=True)
        pltpu.sync_copy(w0b, saccf.at[iw0], add=True)
        pltpu.sync_copy(w1b, saccf.at[iw1], add=True)
        return 0
    lax.fori_loop(0, NCHUNK, chunk_body, 0)

    plsc.subcore_barrier()
    pltpu.sync_copy(acc.at[pl.ds(s * ROWS_PER_TILE, ROWS_PER_TILE)],
                    out_z.at[pl.ds(c * ACC_R + s * ROWS_PER_TILE, ROWS_PER_TILE)])
    pltpu.sync_copy(saccf.at[pl.ds(s * SSLICE, SSLICE)],
                    out_sf.at[pl.ds(c * ACC_R * ALW + s * SSLICE, SSLICE)])


ZFLAT = 2048

_sc_edge = functools.partial(
    pl.kernel,
    out_type=[jax.ShapeDtypeStruct((2 * ACC_R, ZW), jnp.float32),
              jax.ShapeDtypeStruct((2 * ACC_R * ALW,), jnp.float32)],
    mesh=plsc.VectorSubcoreMesh(core_axis_name="c", subcore_axis_name="s"),
    compiler_params=pltpu.CompilerParams(needs_layout_passes=False),
    scratch_types=[
        pltpu.VMEM((CHUNK, ZW), jnp.float32),
        pltpu.VMEM((ZFLAT,), jnp.float32),
        pltpu.VMEM((CHUNK,), jnp.int32),
        pltpu.VMEM((CHUNK,), jnp.int32),
        pltpu.VMEM((CHUNK,), jnp.int32),
        pltpu.VMEM((CHUNK,), jnp.int32),
        pltpu.VMEM((CHUNK,), jnp.int32),
        pltpu.VMEM((CHUNK,), jnp.int32),
        pltpu.VMEM((CHUNK,), jnp.int32),
        pltpu.VMEM((CHUNK,), jnp.float32),
        pltpu.VMEM((CHUNK,), jnp.float32),
        pltpu.VMEM((CHUNK,), jnp.float32),
        pltpu.VMEM((CHUNK,), jnp.float32),
        pltpu.VMEM((CHUNK,), jnp.float32),
        pltpu.VMEM((CHUNK,), jnp.float32),
        pltpu.VMEM((CHUNK,), jnp.int32),
        pltpu.VMEM((CHUNK,), jnp.int32),
        pltpu.VMEM_SHARED((ACC_R * ALW,), jnp.float32),
        pltpu.VMEM_SHARED((ACC_R, ZW), jnp.float32),
        pltpu.VMEM_SHARED((ACC_R * ALW,), jnp.float32),
        pltpu.SemaphoreType.DMA,
    ],
)(_sc_body)


def _write_dense(zcat_ref, al8_ref, zb, a_s, a_d):
    zcat_ref[0] = zb[:, 0:2 * C]
    zcat_ref[1] = zb[:, 2 * C:4 * C]
    cols = []
    for h in range(H):
        cols.append(jnp.sum(zb[:, C * h:C * (h + 1)] * a_s[0:1, h, :],
                            axis=1, keepdims=True))
    for h in range(H):
        cols.append(jnp.sum(zb[:, C * h:C * (h + 1)] * a_d[0:1, h, :],
                            axis=1, keepdims=True))
    cols.append(jnp.zeros((BLK, ALW - 2 * H), jnp.float32))
    al8_ref[...] = jnp.concatenate(cols, axis=1)


def _pre_body(x_ref, w_ref, as_ref, ad_ref, zcat_ref, al8_ref):
    zb = jnp.dot(x_ref[...], w_ref[...], preferred_element_type=jnp.float32)
    _write_dense(zcat_ref, al8_ref, zb, as_ref[...], ad_ref[...])


def _conv_out(y, sden, b):
    eps = jnp.float32(1e-30)
    o = (y[0, :, 0:C] / (sden[0, :, 0:1] + eps)
         + y[0, :, C:2 * C] / (sden[0, :, 1:2] + eps)
         + y[1, :, 0:C] / (sden[1, :, 0:1] + eps)
         + y[1, :, C:2 * C] / (sden[1, :, 1:2] + eps))
    o = o * jnp.float32(0.25) + b
    return jnp.where(o > 0, o, jnp.exp(o) - jnp.float32(1.0))


def _mid_body(y_ref, s_ref, b_ref, hprev_ref, w_ref, as_ref, ad_ref,
              h_ref, zcat_ref, al8_ref):
    h = _conv_out(y_ref[...], s_ref[...], b_ref[...]) + hprev_ref[...]
    h_ref[...] = h
    zb = jnp.dot(h, w_ref[...], preferred_element_type=jnp.float32)
    _write_dense(zcat_ref, al8_ref, zb, as_ref[...], ad_ref[...])


def _post_body(y_ref, s_ref, b_ref, hprev_ref, lw_ref, lb_ref, out_ref):
    h = _conv_out(y_ref[...], s_ref[...], b_ref[...]) + hprev_ref[...]
    out_ref[...] = (jnp.dot(h, lw_ref[...], preferred_element_type=jnp.float32)
                    + lb_ref[...])


def _full(shape):
    return pl.BlockSpec(shape, lambda i: tuple(0 for _ in shape))


def _rows_spec(w):
    return pl.BlockSpec((BLK, w), lambda i: (i, 0))


_zcat_spec = pl.BlockSpec((2, BLK, ZW), lambda i: (0, i, 0))
_y_spec = pl.BlockSpec((2, BLK, ZW), lambda i: (0, i, 0))
_s_spec = pl.BlockSpec((2, BLK, ALW), lambda i: (0, i, 0))

_tc_pre = pl.pallas_call(
    _pre_body,
    grid=(NBLK,),
    in_specs=[_rows_spec(F_IN), _full((F_IN, H * C)),
              _full((1, H, C)), _full((1, H, C))],
    out_specs=[_zcat_spec, _rows_spec(ALW)],
    out_shape=[jax.ShapeDtypeStruct((2, N, ZW), jnp.float32),
               jax.ShapeDtypeStruct((N, ALW), jnp.float32)],
)

_tc_mid = pl.pallas_call(
    _mid_body,
    grid=(NBLK,),
    in_specs=[_y_spec, _s_spec, _full((1, C)), _rows_spec(C),
              _full((C, H * C)), _full((1, H, C)), _full((1, H, C))],
    out_specs=[_rows_spec(C), _zcat_spec, _rows_spec(ALW)],
    out_shape=[jax.ShapeDtypeStruct((N, C), jnp.float32),
               jax.ShapeDtypeStruct((2, N, ZW), jnp.float32),
               jax.ShapeDtypeStruct((N, ALW), jnp.float32)],
)

_tc_post = pl.pallas_call(
    _post_body,
    grid=(NBLK,),
    in_specs=[_y_spec, _s_spec, _full((1, C)), _rows_spec(C), _full((C, 1)),
              _full((1, 1))],
    out_specs=_rows_spec(1),
    out_shape=jax.ShapeDtypeStruct((N, 1), jnp.float32),
)


def kernel(x, edge_index, W1, a_src1, a_dst1, b1, W2, a_src2, a_dst2, b2,
           W3, a_src3, a_dst3, b3, lin_w, lin_b):
    i32 = jnp.int32
    loops = jnp.arange(N, dtype=i32)
    pad = EPAD - ETOT
    srcp = jnp.concatenate([edge_index[0].astype(i32), loops,
                            jnp.zeros((pad,), i32)])
    dstp = jnp.concatenate([edge_index[1].astype(i32), loops,
                            jnp.full((pad,), N, i32)])

    alpad = jnp.zeros((ACC_R - N, ALW), jnp.float32)

    def sc_pass(zcat, al16):
        al16f = jnp.concatenate([al16, alpad]).reshape(ACC_R * ALW)
        yz, ysf = _sc_edge(zcat.reshape(2 * N, ZW), al16f, srcp, dstp)
        return yz.reshape(2, ACC_R, ZW), ysf.reshape(2, ACC_R, ALW)

    zcat1, al81 = _tc_pre(x, W1, a_src1, a_dst1)
    y1, s1 = sc_pass(zcat1, al81)
    h0 = jnp.zeros((N, C), jnp.float32)
    h1, zcat2, al82 = _tc_mid(y1, s1, b1.reshape(1, C), h0, W2, a_src2, a_dst2)
    y2, s2 = sc_pass(zcat2, al82)
    h2, zcat3, al83 = _tc_mid(y2, s2, b2.reshape(1, C), h1, W3, a_src3, a_dst3)
    y3, s3 = sc_pass(zcat3, al83)
    out = _tc_post(y3, s3, b3.reshape(1, C), h2, lin_w, lin_b.reshape(1, 1))
    return out


# speedup vs baseline: 58.1170x; 1.2223x over previous
import functools

import jax
import jax.numpy as jnp
from jax import lax
from jax.experimental import pallas as pl
from jax.experimental.pallas import tpu as pltpu
from jax.experimental.pallas import tpu_sc as plsc

N = 10000
E = 640000
F_IN = 128
H = 4
C = 64

ZW = 128
SW = 16
NTILE = 16
ETOT = E + N
EW = ((ETOT + NTILE * 128 - 1) // (NTILE * 128)) * 128
EPAD = EW * NTILE
CHUNK = 128
NCHUNK = EW // CHUNK
ACC_R = 10240
ROWS_PER_TILE = ACC_R // NTILE
ALW = 16

BLK = 1000
NBLK = N // BLK


def _leaky_exp(e):
    return jnp.exp(jnp.where(e > 0, e, e * jnp.float32(0.2)))


def _sc_body(z_flat, al16f_hbm, srcp, dstp, out_z, out_sf,
             rows, zflat, sidx, gidx, didx,
             ias0, ias1, iad0, iad1, as0b, as1b, ad0b, ad1b,
             w0b, w1b, iw0, iw1, alspf, acc, saccf, gs1, gs2, gs3, gs4, gs5):
    c = lax.axis_index("c")
    s = lax.axis_index("s")
    zf = jnp.zeros((16,), jnp.float32)
    SSLICE = ACC_R * ALW // NTILE

    pltpu.sync_copy(al16f_hbm.at[pl.ds(s * SSLICE, SSLICE)],
                    alspf.at[pl.ds(s * SSLICE, SSLICE)])

    def zrow(i, _):
        for k in range(ZW // 16):
            rows[i, pl.ds(k * 16, 16)] = zf
        return 0
    lax.fori_loop(0, CHUNK, zrow, 0)

    def zflat_body(i, _):
        zflat[pl.ds(i * 16, 16)] = zf
        return 0
    lax.fori_loop(0, ZFLAT // 16, zflat_body, 0)
    for k in range(ROWS_PER_TILE // CHUNK):
        pltpu.sync_copy(rows, acc.at[pl.ds(s * ROWS_PER_TILE + k * CHUNK, CHUNK)])
    for k in range(SSLICE // ZFLAT):
        pltpu.sync_copy(zflat, saccf.at[pl.ds(s * SSLICE + k * ZFLAT, ZFLAT)])
    plsc.subcore_barrier()

    base_e = s * EW
    c_off = c * N
    cs0 = 2 * c
    cs1 = 2 * c + 1
    cd0 = 4 + 2 * c
    cd1 = 5 + 2 * c

    def chunk_body(i, _):
        eb = base_e + i * CHUNK
        c1 = pltpu.async_copy(srcp.at[pl.ds(eb, CHUNK)], sidx, gs1)
        c2 = pltpu.async_copy(dstp.at[pl.ds(eb, CHUNK)], didx, gs2)
        c1.wait()
        c2.wait()
        for j in range(CHUNK // 16):
            dj = pl.ds(j * 16, 16)
            sv = sidx[dj]
            dv = didx[dj]
            sv16 = sv * ALW
            dv16 = dv * ALW
            gidx[dj] = sv + c_off
            ias0[dj] = sv16 + cs0
            ias1[dj] = sv16 + cs1
            iad0[dj] = dv16 + cd0
            iad1[dj] = dv16 + cd1
            iw0[dj] = dv16
            iw1[dj] = dv16 + 1
        g1 = pltpu.async_copy(z_flat.at[gidx], rows, gs1)
        g2 = pltpu.async_copy(alspf.at[ias0], as0b, gs2)
        g3 = pltpu.async_copy(alspf.at[ias1], as1b, gs3)
        g4 = pltpu.async_copy(alspf.at[iad0], ad0b, gs4)
        g5 = pltpu.async_copy(alspf.at[iad1], ad1b, gs5)
        g1.wait()
        g2.wait()
        g3.wait()
        g4.wait()
        g5.wait()

        def grp(j, _):
            dj = pl.ds(j * 16, 16)
            w0 = _leaky_exp(as0b[dj] + ad0b[dj])
            w1 = _leaky_exp(as1b[dj] + ad1b[dj])
            w0b[dj] = w0
            w1b[dj] = w1
            for t in range(16):
                eidx = j * 16 + t
                ev = jnp.full((16,), eidx, jnp.int32)
                w0s = plsc.load_gather(w0b, [ev])
                w1s = plsc.load_gather(w1b, [ev])
                for k in range(4):
                    rows[eidx, pl.ds(k * 16, 16)] = rows[eidx, pl.ds(k * 16, 16)] * w0s
                for k in range(4, 8):
                    rows[eidx, pl.ds(k * 16, 16)] = rows[eidx, pl.ds(k * 16, 16)] * w1s
            return 0
        lax.fori_loop(0, CHUNK // 16, grp, 0)
        pltpu.sync_copy(rows, acc.at[didx], add=True)
        pltpu.sync_copy(w0b, saccf.at[iw0], add=True)
        pltpu.sync_copy(w1b, saccf.at[iw1], add=True)
        return 0
    lax.fori_loop(0, NCHUNK, chunk_body, 0)

    plsc.subcore_barrier()
    pltpu.sync_copy(acc.at[pl.ds(s * ROWS_PER_TILE, ROWS_PER_TILE)],
                    out_z.at[pl.ds(c * ACC_R + s * ROWS_PER_TILE, ROWS_PER_TILE)])
    pltpu.sync_copy(saccf.at[pl.ds(s * SSLICE, SSLICE)],
                    out_sf.at[pl.ds(c * ACC_R * ALW + s * SSLICE, SSLICE)])


ZFLAT = 2048

_sc_edge = functools.partial(
    pl.kernel,
    out_type=[jax.ShapeDtypeStruct((2 * ACC_R, ZW), jnp.float32),
              jax.ShapeDtypeStruct((2 * ACC_R * ALW,), jnp.float32)],
    mesh=plsc.VectorSubcoreMesh(core_axis_name="c", subcore_axis_name="s"),
    compiler_params=pltpu.CompilerParams(needs_layout_passes=False),
    scratch_types=[
        pltpu.VMEM((CHUNK, ZW), jnp.float32),
        pltpu.VMEM((ZFLAT,), jnp.float32),
        pltpu.VMEM((CHUNK,), jnp.int32),
        pltpu.VMEM((CHUNK,), jnp.int32),
        pltpu.VMEM((CHUNK,), jnp.int32),
        pltpu.VMEM((CHUNK,), jnp.int32),
        pltpu.VMEM((CHUNK,), jnp.int32),
        pltpu.VMEM((CHUNK,), jnp.int32),
        pltpu.VMEM((CHUNK,), jnp.int32),
        pltpu.VMEM((CHUNK,), jnp.float32),
        pltpu.VMEM((CHUNK,), jnp.float32),
        pltpu.VMEM((CHUNK,), jnp.float32),
        pltpu.VMEM((CHUNK,), jnp.float32),
        pltpu.VMEM((CHUNK,), jnp.float32),
        pltpu.VMEM((CHUNK,), jnp.float32),
        pltpu.VMEM((CHUNK,), jnp.int32),
        pltpu.VMEM((CHUNK,), jnp.int32),
        pltpu.VMEM_SHARED((ACC_R * ALW,), jnp.float32),
        pltpu.VMEM_SHARED((ACC_R, ZW), jnp.float32),
        pltpu.VMEM_SHARED((ACC_R * ALW,), jnp.float32),
        pltpu.SemaphoreType.DMA,
        pltpu.SemaphoreType.DMA,
        pltpu.SemaphoreType.DMA,
        pltpu.SemaphoreType.DMA,
        pltpu.SemaphoreType.DMA,
    ],
)(_sc_body)


def _write_dense(zcat_ref, al8_ref, zb, a_s, a_d):
    zcat_ref[0] = zb[:, 0:2 * C]
    zcat_ref[1] = zb[:, 2 * C:4 * C]
    cols = []
    for h in range(H):
        cols.append(jnp.sum(zb[:, C * h:C * (h + 1)] * a_s[0:1, h, :],
                            axis=1, keepdims=True))
    for h in range(H):
        cols.append(jnp.sum(zb[:, C * h:C * (h + 1)] * a_d[0:1, h, :],
                            axis=1, keepdims=True))
    cols.append(jnp.zeros((BLK, ALW - 2 * H), jnp.float32))
    al8_ref[...] = jnp.concatenate(cols, axis=1)


def _pre_body(x_ref, w_ref, as_ref, ad_ref, zcat_ref, al8_ref):
    zb = jnp.dot(x_ref[...], w_ref[...], preferred_element_type=jnp.float32)
    _write_dense(zcat_ref, al8_ref, zb, as_ref[...], ad_ref[...])


def _conv_out(y, sden, b):
    eps = jnp.float32(1e-30)
    o = (y[0, :, 0:C] / (sden[0, :, 0:1] + eps)
         + y[0, :, C:2 * C] / (sden[0, :, 1:2] + eps)
         + y[1, :, 0:C] / (sden[1, :, 0:1] + eps)
         + y[1, :, C:2 * C] / (sden[1, :, 1:2] + eps))
    o = o * jnp.float32(0.25) + b
    return jnp.where(o > 0, o, jnp.exp(o) - jnp.float32(1.0))


def _mid_body(y_ref, s_ref, b_ref, hprev_ref, w_ref, as_ref, ad_ref,
              h_ref, zcat_ref, al8_ref):
    h = _conv_out(y_ref[...], s_ref[...], b_ref[...]) + hprev_ref[...]
    h_ref[...] = h
    zb = jnp.dot(h, w_ref[...], preferred_element_type=jnp.float32)
    _write_dense(zcat_ref, al8_ref, zb, as_ref[...], ad_ref[...])


def _post_body(y_ref, s_ref, b_ref, hprev_ref, lw_ref, lb_ref, out_ref):
    h = _conv_out(y_ref[...], s_ref[...], b_ref[...]) + hprev_ref[...]
    out_ref[...] = (jnp.dot(h, lw_ref[...], preferred_element_type=jnp.float32)
                    + lb_ref[...])


def _full(shape):
    return pl.BlockSpec(shape, lambda i: tuple(0 for _ in shape))


def _rows_spec(w):
    return pl.BlockSpec((BLK, w), lambda i: (i, 0))


_zcat_spec = pl.BlockSpec((2, BLK, ZW), lambda i: (0, i, 0))
_y_spec = pl.BlockSpec((2, BLK, ZW), lambda i: (0, i, 0))
_s_spec = pl.BlockSpec((2, BLK, ALW), lambda i: (0, i, 0))

_tc_pre = pl.pallas_call(
    _pre_body,
    grid=(NBLK,),
    in_specs=[_rows_spec(F_IN), _full((F_IN, H * C)),
              _full((1, H, C)), _full((1, H, C))],
    out_specs=[_zcat_spec, _rows_spec(ALW)],
    out_shape=[jax.ShapeDtypeStruct((2, N, ZW), jnp.float32),
               jax.ShapeDtypeStruct((N, ALW), jnp.float32)],
)

_tc_mid = pl.pallas_call(
    _mid_body,
    grid=(NBLK,),
    in_specs=[_y_spec, _s_spec, _full((1, C)), _rows_spec(C),
              _full((C, H * C)), _full((1, H, C)), _full((1, H, C))],
    out_specs=[_rows_spec(C), _zcat_spec, _rows_spec(ALW)],
    out_shape=[jax.ShapeDtypeStruct((N, C), jnp.float32),
               jax.ShapeDtypeStruct((2, N, ZW), jnp.float32),
               jax.ShapeDtypeStruct((N, ALW), jnp.float32)],
)

_tc_post = pl.pallas_call(
    _post_body,
    grid=(NBLK,),
    in_specs=[_y_spec, _s_spec, _full((1, C)), _rows_spec(C), _full((C, 1)),
              _full((1, 1))],
    out_specs=_rows_spec(1),
    out_shape=jax.ShapeDtypeStruct((N, 1), jnp.float32),
)


def kernel(x, edge_index, W1, a_src1, a_dst1, b1, W2, a_src2, a_dst2, b2,
           W3, a_src3, a_dst3, b3, lin_w, lin_b):
    i32 = jnp.int32
    loops = jnp.arange(N, dtype=i32)
    pad = EPAD - ETOT
    srcp = jnp.concatenate([edge_index[0].astype(i32), loops,
                            jnp.zeros((pad,), i32)])
    dstp = jnp.concatenate([edge_index[1].astype(i32), loops,
                            jnp.full((pad,), N, i32)])

    alpad = jnp.zeros((ACC_R - N, ALW), jnp.float32)

    def sc_pass(zcat, al16):
        al16f = jnp.concatenate([al16, alpad]).reshape(ACC_R * ALW)
        yz, ysf = _sc_edge(zcat.reshape(2 * N, ZW), al16f, srcp, dstp)
        return yz.reshape(2, ACC_R, ZW), ysf.reshape(2, ACC_R, ALW)

    zcat1, al81 = _tc_pre(x, W1, a_src1, a_dst1)
    y1, s1 = sc_pass(zcat1, al81)
    h0 = jnp.zeros((N, C), jnp.float32)
    h1, zcat2, al82 = _tc_mid(y1, s1, b1.reshape(1, C), h0, W2, a_src2, a_dst2)
    y2, s2 = sc_pass(zcat2, al82)
    h2, zcat3, al83 = _tc_mid(y2, s2, b2.reshape(1, C), h1, W3, a_src3, a_dst3)
    y3, s3 = sc_pass(zcat3, al83)
    out = _tc_post(y3, s3, b3.reshape(1, C), h2, lin_w, lin_b.reshape(1, 1))
    return out


# speedup vs baseline: 80.2952x; 1.3816x over previous
import functools

import jax
import jax.numpy as jnp
from jax import lax
from jax.experimental import pallas as pl
from jax.experimental.pallas import tpu as pltpu
from jax.experimental.pallas import tpu_sc as plsc

N = 10000
E = 640000
F_IN = 128
H = 4
C = 64

ZW = 128
SW = 16
NTILE = 16
ETOT = E + N
EW = ((ETOT + NTILE * 128 - 1) // (NTILE * 128)) * 128
EPAD = EW * NTILE
CHUNK = 128
NCHUNK = EW // CHUNK
ACC_R = 10240
ROWS_PER_TILE = ACC_R // NTILE
ALW = 16

BLK = 1000
NBLK = N // BLK


def _leaky_exp(e):
    return jnp.exp(jnp.where(e > 0, e, e * jnp.float32(0.2)))


def _sc_body(z_flat, al16f_hbm, srcp, dstp, out_z, out_sf, *refs):
    (rowsA, rowsB, zflat,
     sidxA, gidxA, didxA, ias0A, ias1A, iad0A, iad1A,
     as0A, as1A, ad0A, ad1A, w0A, w1A, iw0A, iw1A,
     sidxB, gidxB, didxB, ias0B, ias1B, iad0B, iad1B,
     as0B, as1B, ad0B, ad1B, w0B, w1B, iw0B, iw1B,
     acc, saccf,
     sA1, sA2, sA3, sA4, sA5, sB1, sB2, sB3, sB4, sB5) = refs
    c = lax.axis_index("c")
    s = lax.axis_index("s")
    zf = jnp.zeros((16,), jnp.float32)
    SSLICE = ACC_R * ALW // NTILE

    def zrow(i, _):
        for k in range(ZW // 16):
            rowsA[i, pl.ds(k * 16, 16)] = zf
        return 0
    lax.fori_loop(0, CHUNK, zrow, 0)

    def zflat_body(i, _):
        zflat[pl.ds(i * 16, 16)] = zf
        return 0
    lax.fori_loop(0, ZFLAT // 16, zflat_body, 0)
    for k in range(ROWS_PER_TILE // CHUNK):
        pltpu.sync_copy(rowsA, acc.at[pl.ds(s * ROWS_PER_TILE + k * CHUNK, CHUNK)])
    for k in range(SSLICE // ZFLAT):
        pltpu.sync_copy(zflat, saccf.at[pl.ds(s * SSLICE + k * ZFLAT, ZFLAT)])
    plsc.subcore_barrier()

    base_e = s * EW
    c_off = c * N
    cs0 = 2 * c
    cs1 = 2 * c + 1
    cd0 = 4 + 2 * c
    cd1 = 5 + 2 * c

    setA = (rowsA, sidxA, gidxA, didxA, ias0A, ias1A, iad0A, iad1A,
            as0A, as1A, ad0A, ad1A, w0A, w1A, iw0A, iw1A,
            sA1, sA2, sA3, sA4, sA5)
    setB = (rowsB, sidxB, gidxB, didxB, ias0B, ias1B, iad0B, iad1B,
            as0B, as1B, ad0B, ad1B, w0B, w1B, iw0B, iw1B,
            sB1, sB2, sB3, sB4, sB5)

    def issue(i, bufs):
        (rows, sidx, gidx, didx, ias0, ias1, iad0, iad1,
         as0b, as1b, ad0b, ad1b, w0b, w1b, iw0, iw1,
         g1s, g2s, g3s, g4s, g5s) = bufs
        eb = base_e + i * CHUNK
        c1 = pltpu.async_copy(srcp.at[pl.ds(eb, CHUNK)], sidx, g1s)
        c2 = pltpu.async_copy(dstp.at[pl.ds(eb, CHUNK)], didx, g2s)
        c1.wait()
        c2.wait()
        for j in range(CHUNK // 16):
            dj = pl.ds(j * 16, 16)
            sv = sidx[dj]
            dv = didx[dj]
            sv16 = sv * ALW
            dv16 = dv * ALW
            gidx[dj] = sv + c_off
            ias0[dj] = sv16 + cs0
            ias1[dj] = sv16 + cs1
            iad0[dj] = dv16 + cd0
            iad1[dj] = dv16 + cd1
            iw0[dj] = dv16
            iw1[dj] = dv16 + 1
        pltpu.async_copy(z_flat.at[gidx], rows, g1s)
        pltpu.async_copy(al16f_hbm.at[ias0], as0b, g2s)
        pltpu.async_copy(al16f_hbm.at[ias1], as1b, g3s)
        pltpu.async_copy(al16f_hbm.at[iad0], ad0b, g4s)
        pltpu.async_copy(al16f_hbm.at[iad1], ad1b, g5s)

    def wait_gathers(bufs):
        (rows, sidx, gidx, didx, ias0, ias1, iad0, iad1,
         as0b, as1b, ad0b, ad1b, w0b, w1b, iw0, iw1,
         g1s, g2s, g3s, g4s, g5s) = bufs
        pltpu.make_async_copy(z_flat.at[gidx], rows, g1s).wait()
        pltpu.make_async_copy(al16f_hbm.at[ias0], as0b, g2s).wait()
        pltpu.make_async_copy(al16f_hbm.at[ias1], as1b, g3s).wait()
        pltpu.make_async_copy(al16f_hbm.at[iad0], ad0b, g4s).wait()
        pltpu.make_async_copy(al16f_hbm.at[iad1], ad1b, g5s).wait()

    def process(bufs):
        (rows, sidx, gidx, didx, ias0, ias1, iad0, iad1,
         as0b, as1b, ad0b, ad1b, w0b, w1b, iw0, iw1,
         g1s, g2s, g3s, g4s, g5s) = bufs

        def grp(j, _):
            dj = pl.ds(j * 16, 16)
            w0 = _leaky_exp(as0b[dj] + ad0b[dj])
            w1 = _leaky_exp(as1b[dj] + ad1b[dj])
            w0b[dj] = w0
            w1b[dj] = w1
            for t in range(16):
                eidx = j * 16 + t
                ev = jnp.full((16,), eidx, jnp.int32)
                w0s = plsc.load_gather(w0b, [ev])
                w1s = plsc.load_gather(w1b, [ev])
                for k in range(4):
                    rows[eidx, pl.ds(k * 16, 16)] = rows[eidx, pl.ds(k * 16, 16)] * w0s
                for k in range(4, 8):
                    rows[eidx, pl.ds(k * 16, 16)] = rows[eidx, pl.ds(k * 16, 16)] * w1s
            return 0
        lax.fori_loop(0, CHUNK // 16, grp, 0)
        pltpu.sync_copy(rows, acc.at[didx], add=True)
        pltpu.sync_copy(w0b, saccf.at[iw0], add=True)
        pltpu.sync_copy(w1b, saccf.at[iw1], add=True)

    PAIRS = NCHUNK // 2
    issue(0, setA)

    def pair_body(k, _):
        issue(2 * k + 1, setB)
        wait_gathers(setA)
        process(setA)

        @pl.when(k < PAIRS - 1)
        def _():
            issue(2 * k + 2, setA)
        wait_gathers(setB)
        process(setB)
        return 0
    lax.fori_loop(0, PAIRS, pair_body, 0)

    plsc.subcore_barrier()
    pltpu.sync_copy(acc.at[pl.ds(s * ROWS_PER_TILE, ROWS_PER_TILE)],
                    out_z.at[pl.ds(c * ACC_R + s * ROWS_PER_TILE, ROWS_PER_TILE)])
    pltpu.sync_copy(saccf.at[pl.ds(s * SSLICE, SSLICE)],
                    out_sf.at[pl.ds(c * ACC_R * ALW + s * SSLICE, SSLICE)])


ZFLAT = 1024

def _chunk_bufs():
    return ([pltpu.VMEM((CHUNK,), jnp.int32)] * 7
            + [pltpu.VMEM((CHUNK,), jnp.float32)] * 6
            + [pltpu.VMEM((CHUNK,), jnp.int32)] * 2)


_sc_edge = functools.partial(
    pl.kernel,
    out_type=[jax.ShapeDtypeStruct((2 * ACC_R, ZW), jnp.float32),
              jax.ShapeDtypeStruct((2 * ACC_R * ALW,), jnp.float32)],
    mesh=plsc.VectorSubcoreMesh(core_axis_name="c", subcore_axis_name="s"),
    compiler_params=pltpu.CompilerParams(needs_layout_passes=False),
    scratch_types=(
        [pltpu.VMEM((CHUNK, ZW), jnp.float32),
         pltpu.VMEM((CHUNK, ZW), jnp.float32),
         pltpu.VMEM((ZFLAT,), jnp.float32)]
        + _chunk_bufs() + _chunk_bufs()
        + [pltpu.VMEM_SHARED((ACC_R, ZW), jnp.float32),
           pltpu.VMEM_SHARED((ACC_R * ALW,), jnp.float32)]
        + [pltpu.SemaphoreType.DMA] * 10
    ),
)(_sc_body)


def _write_dense(zcat_ref, al8_ref, zb, a_s, a_d):
    zcat_ref[0] = zb[:, 0:2 * C]
    zcat_ref[1] = zb[:, 2 * C:4 * C]
    cols = []
    for h in range(H):
        cols.append(jnp.sum(zb[:, C * h:C * (h + 1)] * a_s[0:1, h, :],
                            axis=1, keepdims=True))
    for h in range(H):
        cols.append(jnp.sum(zb[:, C * h:C * (h + 1)] * a_d[0:1, h, :],
                            axis=1, keepdims=True))
    cols.append(jnp.zeros((BLK, ALW - 2 * H), jnp.float32))
    al8_ref[...] = jnp.concatenate(cols, axis=1)


def _pre_body(x_ref, w_ref, as_ref, ad_ref, zcat_ref, al8_ref):
    zb = jnp.dot(x_ref[...], w_ref[...], preferred_element_type=jnp.float32)
    _write_dense(zcat_ref, al8_ref, zb, as_ref[...], ad_ref[...])


def _conv_out(y, sden, b):
    eps = jnp.float32(1e-30)
    o = (y[0, :, 0:C] / (sden[0, :, 0:1] + eps)
         + y[0, :, C:2 * C] / (sden[0, :, 1:2] + eps)
         + y[1, :, 0:C] / (sden[1, :, 0:1] + eps)
         + y[1, :, C:2 * C] / (sden[1, :, 1:2] + eps))
    o = o * jnp.float32(0.25) + b
    return jnp.where(o > 0, o, jnp.exp(o) - jnp.float32(1.0))


def _mid_body(y_ref, s_ref, b_ref, hprev_ref, w_ref, as_ref, ad_ref,
              h_ref, zcat_ref, al8_ref):
    h = _conv_out(y_ref[...], s_ref[...], b_ref[...]) + hprev_ref[...]
    h_ref[...] = h
    zb = jnp.dot(h, w_ref[...], preferred_element_type=jnp.float32)
    _write_dense(zcat_ref, al8_ref, zb, as_ref[...], ad_ref[...])


def _post_body(y_ref, s_ref, b_ref, hprev_ref, lw_ref, lb_ref, out_ref):
    h = _conv_out(y_ref[...], s_ref[...], b_ref[...]) + hprev_ref[...]
    out_ref[...] = (jnp.dot(h, lw_ref[...], preferred_element_type=jnp.float32)
                    + lb_ref[...])


def _full(shape):
    return pl.BlockSpec(shape, lambda i: tuple(0 for _ in shape))


def _rows_spec(w):
    return pl.BlockSpec((BLK, w), lambda i: (i, 0))


_zcat_spec = pl.BlockSpec((2, BLK, ZW), lambda i: (0, i, 0))
_y_spec = pl.BlockSpec((2, BLK, ZW), lambda i: (0, i, 0))
_s_spec = pl.BlockSpec((2, BLK, ALW), lambda i: (0, i, 0))

_tc_pre = pl.pallas_call(
    _pre_body,
    grid=(NBLK,),
    in_specs=[_rows_spec(F_IN), _full((F_IN, H * C)),
              _full((1, H, C)), _full((1, H, C))],
    out_specs=[_zcat_spec, _rows_spec(ALW)],
    out_shape=[jax.ShapeDtypeStruct((2, N, ZW), jnp.float32),
               jax.ShapeDtypeStruct((N, ALW), jnp.float32)],
)

_tc_mid = pl.pallas_call(
    _mid_body,
    grid=(NBLK,),
    in_specs=[_y_spec, _s_spec, _full((1, C)), _rows_spec(C),
              _full((C, H * C)), _full((1, H, C)), _full((1, H, C))],
    out_specs=[_rows_spec(C), _zcat_spec, _rows_spec(ALW)],
    out_shape=[jax.ShapeDtypeStruct((N, C), jnp.float32),
               jax.ShapeDtypeStruct((2, N, ZW), jnp.float32),
               jax.ShapeDtypeStruct((N, ALW), jnp.float32)],
)

_tc_post = pl.pallas_call(
    _post_body,
    grid=(NBLK,),
    in_specs=[_y_spec, _s_spec, _full((1, C)), _rows_spec(C), _full((C, 1)),
              _full((1, 1))],
    out_specs=_rows_spec(1),
    out_shape=jax.ShapeDtypeStruct((N, 1), jnp.float32),
)


def kernel(x, edge_index, W1, a_src1, a_dst1, b1, W2, a_src2, a_dst2, b2,
           W3, a_src3, a_dst3, b3, lin_w, lin_b):
    i32 = jnp.int32
    loops = jnp.arange(N, dtype=i32)
    pad = EPAD - ETOT
    srcp = jnp.concatenate([edge_index[0].astype(i32), loops,
                            jnp.zeros((pad,), i32)])
    dstp = jnp.concatenate([edge_index[1].astype(i32), loops,
                            jnp.full((pad,), N, i32)])

    alpad = jnp.zeros((ACC_R - N, ALW), jnp.float32)

    def sc_pass(zcat, al16):
        al16f = jnp.concatenate([al16, alpad]).reshape(ACC_R * ALW)
        yz, ysf = _sc_edge(zcat.reshape(2 * N, ZW), al16f, srcp, dstp)
        return yz.reshape(2, ACC_R, ZW), ysf.reshape(2, ACC_R, ALW)

    zcat1, al81 = _tc_pre(x, W1, a_src1, a_dst1)
    y1, s1 = sc_pass(zcat1, al81)
    h0 = jnp.zeros((N, C), jnp.float32)
    h1, zcat2, al82 = _tc_mid(y1, s1, b1.reshape(1, C), h0, W2, a_src2, a_dst2)
    y2, s2 = sc_pass(zcat2, al82)
    h2, zcat3, al83 = _tc_mid(y2, s2, b2.reshape(1, C), h1, W3, a_src3, a_dst3)
    y3, s3 = sc_pass(zcat3, al83)
    out = _tc_post(y3, s3, b3.reshape(1, C), h2, lin_w, lin_b.reshape(1, 1))
    return out


# speedup vs baseline: 83.1666x; 1.0358x over previous
import functools

import jax
import jax.numpy as jnp
from jax import lax
from jax.experimental import pallas as pl
from jax.experimental.pallas import tpu as pltpu
from jax.experimental.pallas import tpu_sc as plsc

N = 10000
E = 640000
F_IN = 128
H = 4
C = 64

ZW = 128
SW = 16
NTILE = 16
ETOT = E + N
EW = ((ETOT + NTILE * 128 - 1) // (NTILE * 128)) * 128
EPAD = EW * NTILE
CHUNK = 128
NCHUNK = EW // CHUNK
ACC_R = 10240
ROWS_PER_TILE = ACC_R // NTILE
ALW = 16

BLK = 1000
NBLK = N // BLK


def _leaky_exp(e):
    return jnp.exp(jnp.where(e > 0, e, e * jnp.float32(0.2)))


def _sc_body(z_flat, al16f_hbm, srcp, dstp, out_z, out_sf, *refs):
    (rowsA, rowsB, zflat,
     sidxA, gidxA, didxA, ias0A, ias1A, iad0A, iad1A,
     as0A, as1A, ad0A, ad1A, w0A, w1A, iw0A, iw1A,
     sidxB, gidxB, didxB, ias0B, ias1B, iad0B, iad1B,
     as0B, as1B, ad0B, ad1B, w0B, w1B, iw0B, iw1B,
     acc, saccf,
     sA1, sA2, sA3, sA4, sA5, tA1, tA2, tA3,
     sB1, sB2, sB3, sB4, sB5, tB1, tB2, tB3) = refs
    c = lax.axis_index("c")
    s = lax.axis_index("s")
    zf = jnp.zeros((16,), jnp.float32)
    SSLICE = ACC_R * ALW // NTILE

    def zrow(i, _):
        for k in range(ZW // 16):
            rowsA[i, pl.ds(k * 16, 16)] = zf
        return 0
    lax.fori_loop(0, CHUNK, zrow, 0)

    def zflat_body(i, _):
        zflat[pl.ds(i * 16, 16)] = zf
        return 0
    lax.fori_loop(0, ZFLAT // 16, zflat_body, 0)
    for k in range(ROWS_PER_TILE // CHUNK):
        pltpu.sync_copy(rowsA, acc.at[pl.ds(s * ROWS_PER_TILE + k * CHUNK, CHUNK)])
    for k in range(SSLICE // ZFLAT):
        pltpu.sync_copy(zflat, saccf.at[pl.ds(s * SSLICE + k * ZFLAT, ZFLAT)])
    plsc.subcore_barrier()

    base_e = s * EW
    c_off = c * N
    cs0 = 2 * c
    cs1 = 2 * c + 1
    cd0 = 4 + 2 * c
    cd1 = 5 + 2 * c

    setA = (rowsA, sidxA, gidxA, didxA, ias0A, ias1A, iad0A, iad1A,
            as0A, as1A, ad0A, ad1A, w0A, w1A, iw0A, iw1A,
            sA1, sA2, sA3, sA4, sA5, tA1, tA2, tA3)
    setB = (rowsB, sidxB, gidxB, didxB, ias0B, ias1B, iad0B, iad1B,
            as0B, as1B, ad0B, ad1B, w0B, w1B, iw0B, iw1B,
            sB1, sB2, sB3, sB4, sB5, tB1, tB2, tB3)

    def issue(i, bufs):
        (rows, sidx, gidx, didx, ias0, ias1, iad0, iad1,
         as0b, as1b, ad0b, ad1b, w0b, w1b, iw0, iw1,
         g1s, g2s, g3s, g4s, g5s, t1s, t2s, t3s) = bufs
        eb = base_e + i * CHUNK
        c1 = pltpu.async_copy(srcp.at[pl.ds(eb, CHUNK)], sidx, g1s)
        c2 = pltpu.async_copy(dstp.at[pl.ds(eb, CHUNK)], didx, g2s)
        c1.wait()
        c2.wait()
        for j in range(CHUNK // 16):
            dj = pl.ds(j * 16, 16)
            sv = sidx[dj]
            dv = didx[dj]
            sv16 = sv * ALW
            dv16 = dv * ALW
            gidx[dj] = sv + c_off
            ias0[dj] = sv16 + cs0
            ias1[dj] = sv16 + cs1
            iad0[dj] = dv16 + cd0
            iad1[dj] = dv16 + cd1
            iw0[dj] = dv16
            iw1[dj] = dv16 + 1
        pltpu.async_copy(z_flat.at[gidx], rows, g1s)
        pltpu.async_copy(al16f_hbm.at[ias0], as0b, g2s)
        pltpu.async_copy(al16f_hbm.at[ias1], as1b, g3s)
        pltpu.async_copy(al16f_hbm.at[iad0], ad0b, g4s)
        pltpu.async_copy(al16f_hbm.at[iad1], ad1b, g5s)

    def wait_gathers(bufs):
        (rows, sidx, gidx, didx, ias0, ias1, iad0, iad1,
         as0b, as1b, ad0b, ad1b, w0b, w1b, iw0, iw1,
         g1s, g2s, g3s, g4s, g5s, t1s, t2s, t3s) = bufs
        pltpu.make_async_copy(z_flat.at[gidx], rows, g1s).wait()
        pltpu.make_async_copy(al16f_hbm.at[ias0], as0b, g2s).wait()
        pltpu.make_async_copy(al16f_hbm.at[ias1], as1b, g3s).wait()
        pltpu.make_async_copy(al16f_hbm.at[iad0], ad0b, g4s).wait()
        pltpu.make_async_copy(al16f_hbm.at[iad1], ad1b, g5s).wait()

    def process(bufs):
        (rows, sidx, gidx, didx, ias0, ias1, iad0, iad1,
         as0b, as1b, ad0b, ad1b, w0b, w1b, iw0, iw1,
         g1s, g2s, g3s, g4s, g5s, t1s, t2s, t3s) = bufs

        def grp(j, _):
            dj = pl.ds(j * 16, 16)
            w0 = _leaky_exp(as0b[dj] + ad0b[dj])
            w1 = _leaky_exp(as1b[dj] + ad1b[dj])
            w0b[dj] = w0
            w1b[dj] = w1
            for t in range(16):
                eidx = j * 16 + t
                ev = jnp.full((16,), eidx, jnp.int32)
                w0s = plsc.load_gather(w0b, [ev])
                w1s = plsc.load_gather(w1b, [ev])
                for k in range(4):
                    rows[eidx, pl.ds(k * 16, 16)] = rows[eidx, pl.ds(k * 16, 16)] * w0s
                for k in range(4, 8):
                    rows[eidx, pl.ds(k * 16, 16)] = rows[eidx, pl.ds(k * 16, 16)] * w1s
            return 0
        lax.fori_loop(0, CHUNK // 16, grp, 0)
        x1 = pltpu.async_copy(rows, acc.at[didx], t1s, add=True)
        x2 = pltpu.async_copy(w0b, saccf.at[iw0], t2s, add=True)
        x3 = pltpu.async_copy(w1b, saccf.at[iw1], t3s, add=True)
        x1.wait()
        x2.wait()
        x3.wait()

    PAIRS = NCHUNK // 2
    issue(0, setA)

    def pair_body(k, _):
        issue(2 * k + 1, setB)
        wait_gathers(setA)
        process(setA)

        @pl.when(k < PAIRS - 1)
        def _():
            issue(2 * k + 2, setA)
        wait_gathers(setB)
        process(setB)
        return 0
    lax.fori_loop(0, PAIRS, pair_body, 0)

    plsc.subcore_barrier()
    pltpu.sync_copy(acc.at[pl.ds(s * ROWS_PER_TILE, ROWS_PER_TILE)],
                    out_z.at[pl.ds(c * ACC_R + s * ROWS_PER_TILE, ROWS_PER_TILE)])
    pltpu.sync_copy(saccf.at[pl.ds(s * SSLICE, SSLICE)],
                    out_sf.at[pl.ds(c * ACC_R * ALW + s * SSLICE, SSLICE)])


ZFLAT = 1024

def _chunk_bufs():
    return ([pltpu.VMEM((CHUNK,), jnp.int32)] * 7
            + [pltpu.VMEM((CHUNK,), jnp.float32)] * 6
            + [pltpu.VMEM((CHUNK,), jnp.int32)] * 2)


_sc_edge = functools.partial(
    pl.kernel,
    out_type=[jax.ShapeDtypeStruct((2 * ACC_R, ZW), jnp.float32),
              jax.ShapeDtypeStruct((2 * ACC_R * ALW,), jnp.float32)],
    mesh=plsc.VectorSubcoreMesh(core_axis_name="c", subcore_axis_name="s"),
    compiler_params=pltpu.CompilerParams(needs_layout_passes=False),
    scratch_types=(
        [pltpu.VMEM((CHUNK, ZW), jnp.float32),
         pltpu.VMEM((CHUNK, ZW), jnp.float32),
         pltpu.VMEM((ZFLAT,), jnp.float32)]
        + _chunk_bufs() + _chunk_bufs()
        + [pltpu.VMEM_SHARED((ACC_R, ZW), jnp.float32),
           pltpu.VMEM_SHARED((ACC_R * ALW,), jnp.float32)]
        + [pltpu.SemaphoreType.DMA] * 16
    ),
)(_sc_body)


def _write_dense(zcat_ref, al8_ref, zb, a_s, a_d):
    zcat_ref[0] = zb[:, 0:2 * C]
    zcat_ref[1] = zb[:, 2 * C:4 * C]
    cols = []
    for h in range(H):
        cols.append(jnp.sum(zb[:, C * h:C * (h + 1)] * a_s[0:1, h, :],
                            axis=1, keepdims=True))
    for h in range(H):
        cols.append(jnp.sum(zb[:, C * h:C * (h + 1)] * a_d[0:1, h, :],
                            axis=1, keepdims=True))
    cols.append(jnp.zeros((BLK, ALW - 2 * H), jnp.float32))
    al8_ref[...] = jnp.concatenate(cols, axis=1)


def _pre_body(x_ref, w_ref, as_ref, ad_ref, zcat_ref, al8_ref):
    zb = jnp.dot(x_ref[...], w_ref[...], preferred_element_type=jnp.float32)
    _write_dense(zcat_ref, al8_ref, zb, as_ref[...], ad_ref[...])


def _conv_out(y, sden, b):
    eps = jnp.float32(1e-30)
    o = (y[0, :, 0:C] / (sden[0, :, 0:1] + eps)
         + y[0, :, C:2 * C] / (sden[0, :, 1:2] + eps)
         + y[1, :, 0:C] / (sden[1, :, 0:1] + eps)
         + y[1, :, C:2 * C] / (sden[1, :, 1:2] + eps))
    o = o * jnp.float32(0.25) + b
    return jnp.where(o > 0, o, jnp.exp(o) - jnp.float32(1.0))


def _mid_body(y_ref, s_ref, b_ref, hprev_ref, w_ref, as_ref, ad_ref,
              h_ref, zcat_ref, al8_ref):
    h = _conv_out(y_ref[...], s_ref[...], b_ref[...]) + hprev_ref[...]
    h_ref[...] = h
    zb = jnp.dot(h, w_ref[...], preferred_element_type=jnp.float32)
    _write_dense(zcat_ref, al8_ref, zb, as_ref[...], ad_ref[...])


def _post_body(y_ref, s_ref, b_ref, hprev_ref, lw_ref, lb_ref, out_ref):
    h = _conv_out(y_ref[...], s_ref[...], b_ref[...]) + hprev_ref[...]
    out_ref[...] = (jnp.dot(h, lw_ref[...], preferred_element_type=jnp.float32)
                    + lb_ref[...])


def _full(shape):
    return pl.BlockSpec(shape, lambda i: tuple(0 for _ in shape))


def _rows_spec(w):
    return pl.BlockSpec((BLK, w), lambda i: (i, 0))


_zcat_spec = pl.BlockSpec((2, BLK, ZW), lambda i: (0, i, 0))
_y_spec = pl.BlockSpec((2, BLK, ZW), lambda i: (0, i, 0))
_s_spec = pl.BlockSpec((2, BLK, ALW), lambda i: (0, i, 0))

_tc_pre = pl.pallas_call(
    _pre_body,
    grid=(NBLK,),
    in_specs=[_rows_spec(F_IN), _full((F_IN, H * C)),
              _full((1, H, C)), _full((1, H, C))],
    out_specs=[_zcat_spec, _rows_spec(ALW)],
    out_shape=[jax.ShapeDtypeStruct((2, N, ZW), jnp.float32),
               jax.ShapeDtypeStruct((N, ALW), jnp.float32)],
)

_tc_mid = pl.pallas_call(
    _mid_body,
    grid=(NBLK,),
    in_specs=[_y_spec, _s_spec, _full((1, C)), _rows_spec(C),
              _full((C, H * C)), _full((1, H, C)), _full((1, H, C))],
    out_specs=[_rows_spec(C), _zcat_spec, _rows_spec(ALW)],
    out_shape=[jax.ShapeDtypeStruct((N, C), jnp.float32),
               jax.ShapeDtypeStruct((2, N, ZW), jnp.float32),
               jax.ShapeDtypeStruct((N, ALW), jnp.float32)],
)

_tc_post = pl.pallas_call(
    _post_body,
    grid=(NBLK,),
    in_specs=[_y_spec, _s_spec, _full((1, C)), _rows_spec(C), _full((C, 1)),
              _full((1, 1))],
    out_specs=_rows_spec(1),
    out_shape=jax.ShapeDtypeStruct((N, 1), jnp.float32),
)


def kernel(x, edge_index, W1, a_src1, a_dst1, b1, W2, a_src2, a_dst2, b2,
           W3, a_src3, a_dst3, b3, lin_w, lin_b):
    i32 = jnp.int32
    loops = jnp.arange(N, dtype=i32)
    pad = EPAD - ETOT
    srcp = jnp.concatenate([edge_index[0].astype(i32), loops,
                            jnp.zeros((pad,), i32)])
    dstp = jnp.concatenate([edge_index[1].astype(i32), loops,
                            jnp.full((pad,), N, i32)])

    alpad = jnp.zeros((ACC_R - N, ALW), jnp.float32)

    def sc_pass(zcat, al16):
        al16f = jnp.concatenate([al16, alpad]).reshape(ACC_R * ALW)
        yz, ysf = _sc_edge(zcat.reshape(2 * N, ZW), al16f, srcp, dstp)
        return yz.reshape(2, ACC_R, ZW), ysf.reshape(2, ACC_R, ALW)

    zcat1, al81 = _tc_pre(x, W1, a_src1, a_dst1)
    y1, s1 = sc_pass(zcat1, al81)
    h0 = jnp.zeros((N, C), jnp.float32)
    h1, zcat2, al82 = _tc_mid(y1, s1, b1.reshape(1, C), h0, W2, a_src2, a_dst2)
    y2, s2 = sc_pass(zcat2, al82)
    h2, zcat3, al83 = _tc_mid(y2, s2, b2.reshape(1, C), h1, W3, a_src3, a_dst3)
    y3, s3 = sc_pass(zcat3, al83)
    out = _tc_post(y3, s3, b3.reshape(1, C), h2, lin_w, lin_b.reshape(1, 1))
    return out


# speedup vs baseline: 123.5764x; 1.4859x over previous
import functools

import jax
import jax.numpy as jnp
from jax import lax
from jax.experimental import pallas as pl
from jax.experimental.pallas import tpu as pltpu
from jax.experimental.pallas import tpu_sc as plsc

N = 10000
E = 640000
F_IN = 128
H = 4
C = 64

ZW = 128
SW = 16
NTILE = 16
ETOT = E + N
EW = ((ETOT + NTILE * 128 - 1) // (NTILE * 128)) * 128
EPAD = EW * NTILE
CHUNK = 128
NCHUNK = EW // CHUNK
ACC_R = 10240
ROWS_PER_TILE = ACC_R // NTILE
ALW = 16

BLK = 1000
NBLK = N // BLK


def _leaky_exp(e):
    return jnp.exp(jnp.where(e > 0, e, e * jnp.float32(0.2)))


def _sc_body(z_flat, al16f_hbm, srcp, dstp, out_z, out_sf, *refs):
    (rowsA, rowsB, zflat,
     psrcA, pdstA, gidxA, didxA, ias0A, ias1A, iad0A, iad1A,
     as0A, as1A, ad0A, ad1A, w0A, w1A, iw0A, iw1A,
     psrcB, pdstB, gidxB, didxB, ias0B, ias1B, iad0B, iad1B,
     as0B, as1B, ad0B, ad1B, w0B, w1B, iw0B, iw1B,
     acc, saccf,
     sA1, sA2, sA3, sA4, sA5, tA1, tA2, tA3, pA1, pA2,
     sB1, sB2, sB3, sB4, sB5, tB1, tB2, tB3, pB1, pB2) = refs
    c = lax.axis_index("c")
    s = lax.axis_index("s")
    zf = jnp.zeros((16,), jnp.float32)
    SSLICE = ACC_R * ALW // NTILE

    def zrow(i, _):
        for k in range(ZW // 16):
            rowsA[i, pl.ds(k * 16, 16)] = zf
        return 0
    lax.fori_loop(0, CHUNK, zrow, 0)

    def zflat_body(i, _):
        zflat[pl.ds(i * 16, 16)] = zf
        return 0
    lax.fori_loop(0, ZFLAT // 16, zflat_body, 0)
    for k in range(ROWS_PER_TILE // CHUNK):
        pltpu.sync_copy(rowsA, acc.at[pl.ds(s * ROWS_PER_TILE + k * CHUNK, CHUNK)])
    for k in range(SSLICE // ZFLAT):
        pltpu.sync_copy(zflat, saccf.at[pl.ds(s * SSLICE + k * ZFLAT, ZFLAT)])
    plsc.subcore_barrier()

    base_e = s * EW
    c_off = c * N
    cs0 = 2 * c
    cs1 = 2 * c + 1
    cd0 = 4 + 2 * c
    cd1 = 5 + 2 * c

    setA = (rowsA, psrcA, pdstA, gidxA, didxA, ias0A, ias1A, iad0A, iad1A,
            as0A, as1A, ad0A, ad1A, w0A, w1A, iw0A, iw1A,
            sA1, sA2, sA3, sA4, sA5, tA1, tA2, tA3, pA1, pA2)
    setB = (rowsB, psrcB, pdstB, gidxB, didxB, ias0B, ias1B, iad0B, iad1B,
            as0B, as1B, ad0B, ad1B, w0B, w1B, iw0B, iw1B,
            sB1, sB2, sB3, sB4, sB5, tB1, tB2, tB3, pB1, pB2)

    def prefetch(i, bufs):
        (rows, psrc, pdst, gidx, didx, ias0, ias1, iad0, iad1,
         as0b, as1b, ad0b, ad1b, w0b, w1b, iw0, iw1,
         g1s, g2s, g3s, g4s, g5s, t1s, t2s, t3s, p1s, p2s) = bufs
        eb = base_e + i * CHUNK
        pltpu.async_copy(srcp.at[pl.ds(eb, CHUNK)], psrc, p1s)
        pltpu.async_copy(dstp.at[pl.ds(eb, CHUNK)], pdst, p2s)

    def issue(i, bufs):
        (rows, psrc, pdst, gidx, didx, ias0, ias1, iad0, iad1,
         as0b, as1b, ad0b, ad1b, w0b, w1b, iw0, iw1,
         g1s, g2s, g3s, g4s, g5s, t1s, t2s, t3s, p1s, p2s) = bufs
        eb = base_e + i * CHUNK
        pltpu.make_async_copy(srcp.at[pl.ds(eb, CHUNK)], psrc, p1s).wait()
        pltpu.make_async_copy(dstp.at[pl.ds(eb, CHUNK)], pdst, p2s).wait()
        for j in range(CHUNK // 16):
            dj = pl.ds(j * 16, 16)
            sv = psrc[dj]
            dv = pdst[dj]
            sv16 = sv * ALW
            dv16 = dv * ALW
            gidx[dj] = sv + c_off
            didx[dj] = dv
            ias0[dj] = sv16 + cs0
            ias1[dj] = sv16 + cs1
            iad0[dj] = dv16 + cd0
            iad1[dj] = dv16 + cd1
            iw0[dj] = dv16
            iw1[dj] = dv16 + 1
        pltpu.async_copy(z_flat.at[gidx], rows, g1s)
        pltpu.async_copy(al16f_hbm.at[ias0], as0b, g2s)
        pltpu.async_copy(al16f_hbm.at[ias1], as1b, g3s)
        pltpu.async_copy(al16f_hbm.at[iad0], ad0b, g4s)
        pltpu.async_copy(al16f_hbm.at[iad1], ad1b, g5s)

        @pl.when(i + 2 < NCHUNK)
        def _():
            prefetch(i + 2, bufs)

    def wait_gathers(bufs):
        (rows, psrc, pdst, gidx, didx, ias0, ias1, iad0, iad1,
         as0b, as1b, ad0b, ad1b, w0b, w1b, iw0, iw1,
         g1s, g2s, g3s, g4s, g5s, t1s, t2s, t3s, p1s, p2s) = bufs
        pltpu.make_async_copy(z_flat.at[gidx], rows, g1s).wait()
        pltpu.make_async_copy(al16f_hbm.at[ias0], as0b, g2s).wait()
        pltpu.make_async_copy(al16f_hbm.at[ias1], as1b, g3s).wait()
        pltpu.make_async_copy(al16f_hbm.at[iad0], ad0b, g4s).wait()
        pltpu.make_async_copy(al16f_hbm.at[iad1], ad1b, g5s).wait()

    def process(bufs):
        (rows, psrc, pdst, gidx, didx, ias0, ias1, iad0, iad1,
         as0b, as1b, ad0b, ad1b, w0b, w1b, iw0, iw1,
         g1s, g2s, g3s, g4s, g5s, t1s, t2s, t3s, p1s, p2s) = bufs

        def grp(j, _):
            dj = pl.ds(j * 16, 16)
            w0 = _leaky_exp(as0b[dj] + ad0b[dj])
            w1 = _leaky_exp(as1b[dj] + ad1b[dj])
            w0b[dj] = w0
            w1b[dj] = w1
            for t in range(16):
                eidx = j * 16 + t
                ev = jnp.full((16,), t, jnp.int32)
                w0s = jnp.take(w0, ev)
                w1s = jnp.take(w1, ev)
                for k in range(4):
                    rows[eidx, pl.ds(k * 16, 16)] = rows[eidx, pl.ds(k * 16, 16)] * w0s
                for k in range(4, 8):
                    rows[eidx, pl.ds(k * 16, 16)] = rows[eidx, pl.ds(k * 16, 16)] * w1s
            return 0
        lax.fori_loop(0, CHUNK // 16, grp, 0)
        x1 = pltpu.async_copy(rows, acc.at[didx], t1s, add=True)
        x2 = pltpu.async_copy(w0b, saccf.at[iw0], t2s, add=True)
        x3 = pltpu.async_copy(w1b, saccf.at[iw1], t3s, add=True)
        x1.wait()
        x2.wait()
        x3.wait()

    PAIRS = NCHUNK // 2
    prefetch(0, setA)
    prefetch(1, setB)
    issue(0, setA)

    def pair_body(k, _):
        issue(2 * k + 1, setB)
        wait_gathers(setA)
        process(setA)

        @pl.when(k < PAIRS - 1)
        def _():
            issue(2 * k + 2, setA)
        wait_gathers(setB)
        process(setB)
        return 0
    lax.fori_loop(0, PAIRS, pair_body, 0)

    plsc.subcore_barrier()
    pltpu.sync_copy(acc.at[pl.ds(s * ROWS_PER_TILE, ROWS_PER_TILE)],
                    out_z.at[pl.ds(c * ACC_R + s * ROWS_PER_TILE, ROWS_PER_TILE)])
    pltpu.sync_copy(saccf.at[pl.ds(s * SSLICE, SSLICE)],
                    out_sf.at[pl.ds(c * ACC_R * ALW + s * SSLICE, SSLICE)])


ZFLAT = 1024

def _chunk_bufs():
    return ([pltpu.VMEM((CHUNK,), jnp.int32)] * 8
            + [pltpu.VMEM((CHUNK,), jnp.float32)] * 6
            + [pltpu.VMEM((CHUNK,), jnp.int32)] * 2)


_sc_edge = functools.partial(
    pl.kernel,
    out_type=[jax.ShapeDtypeStruct((2 * ACC_R, ZW), jnp.float32),
              jax.ShapeDtypeStruct((2 * ACC_R * ALW,), jnp.float32)],
    mesh=plsc.VectorSubcoreMesh(core_axis_name="c", subcore_axis_name="s"),
    compiler_params=pltpu.CompilerParams(needs_layout_passes=False),
    scratch_types=(
        [pltpu.VMEM((CHUNK, ZW), jnp.float32),
         pltpu.VMEM((CHUNK, ZW), jnp.float32),
         pltpu.VMEM((ZFLAT,), jnp.float32)]
        + _chunk_bufs() + _chunk_bufs()
        + [pltpu.VMEM_SHARED((ACC_R, ZW), jnp.float32),
           pltpu.VMEM_SHARED((ACC_R * ALW,), jnp.float32)]
        + [pltpu.SemaphoreType.DMA] * 20
    ),
)(_sc_body)


def _write_dense(zcat_ref, al8_ref, zb, a_s, a_d):
    zcat_ref[0] = zb[:, 0:2 * C]
    zcat_ref[1] = zb[:, 2 * C:4 * C]
    cols = []
    for h in range(H):
        cols.append(jnp.sum(zb[:, C * h:C * (h + 1)] * a_s[0:1, h, :],
                            axis=1, keepdims=True))
    for h in range(H):
        cols.append(jnp.sum(zb[:, C * h:C * (h + 1)] * a_d[0:1, h, :],
                            axis=1, keepdims=True))
    cols.append(jnp.zeros((BLK, ALW - 2 * H), jnp.float32))
    al8_ref[...] = jnp.concatenate(cols, axis=1)


def _pre_body(x_ref, w_ref, as_ref, ad_ref, zcat_ref, al8_ref):
    zb = jnp.dot(x_ref[...], w_ref[...], preferred_element_type=jnp.float32)
    _write_dense(zcat_ref, al8_ref, zb, as_ref[...], ad_ref[...])


def _conv_out(y, sden, b):
    eps = jnp.float32(1e-30)
    o = (y[0, :, 0:C] / (sden[0, :, 0:1] + eps)
         + y[0, :, C:2 * C] / (sden[0, :, 1:2] + eps)
         + y[1, :, 0:C] / (sden[1, :, 0:1] + eps)
         + y[1, :, C:2 * C] / (sden[1, :, 1:2] + eps))
    o = o * jnp.float32(0.25) + b
    return jnp.where(o > 0, o, jnp.exp(o) - jnp.float32(1.0))


def _mid_body(y_ref, s_ref, b_ref, hprev_ref, w_ref, as_ref, ad_ref,
              h_ref, zcat_ref, al8_ref):
    h = _conv_out(y_ref[...], s_ref[...], b_ref[...]) + hprev_ref[...]
    h_ref[...] = h
    zb = jnp.dot(h, w_ref[...], preferred_element_type=jnp.float32)
    _write_dense(zcat_ref, al8_ref, zb, as_ref[...], ad_ref[...])


def _post_body(y_ref, s_ref, b_ref, hprev_ref, lw_ref, lb_ref, out_ref):
    h = _conv_out(y_ref[...], s_ref[...], b_ref[...]) + hprev_ref[...]
    out_ref[...] = (jnp.dot(h, lw_ref[...], preferred_element_type=jnp.float32)
                    + lb_ref[...])


def _full(shape):
    return pl.BlockSpec(shape, lambda i: tuple(0 for _ in shape))


def _rows_spec(w):
    return pl.BlockSpec((BLK, w), lambda i: (i, 0))


_zcat_spec = pl.BlockSpec((2, BLK, ZW), lambda i: (0, i, 0))
_y_spec = pl.BlockSpec((2, BLK, ZW), lambda i: (0, i, 0))
_s_spec = pl.BlockSpec((2, BLK, ALW), lambda i: (0, i, 0))

_tc_pre = pl.pallas_call(
    _pre_body,
    grid=(NBLK,),
    in_specs=[_rows_spec(F_IN), _full((F_IN, H * C)),
              _full((1, H, C)), _full((1, H, C))],
    out_specs=[_zcat_spec, _rows_spec(ALW)],
    out_shape=[jax.ShapeDtypeStruct((2, N, ZW), jnp.float32),
               jax.ShapeDtypeStruct((N, ALW), jnp.float32)],
)

_tc_mid = pl.pallas_call(
    _mid_body,
    grid=(NBLK,),
    in_specs=[_y_spec, _s_spec, _full((1, C)), _rows_spec(C),
              _full((C, H * C)), _full((1, H, C)), _full((1, H, C))],
    out_specs=[_rows_spec(C), _zcat_spec, _rows_spec(ALW)],
    out_shape=[jax.ShapeDtypeStruct((N, C), jnp.float32),
               jax.ShapeDtypeStruct((2, N, ZW), jnp.float32),
               jax.ShapeDtypeStruct((N, ALW), jnp.float32)],
)

_tc_post = pl.pallas_call(
    _post_body,
    grid=(NBLK,),
    in_specs=[_y_spec, _s_spec, _full((1, C)), _rows_spec(C), _full((C, 1)),
              _full((1, 1))],
    out_specs=_rows_spec(1),
    out_shape=jax.ShapeDtypeStruct((N, 1), jnp.float32),
)


def kernel(x, edge_index, W1, a_src1, a_dst1, b1, W2, a_src2, a_dst2, b2,
           W3, a_src3, a_dst3, b3, lin_w, lin_b):
    i32 = jnp.int32
    loops = jnp.arange(N, dtype=i32)
    pad = EPAD - ETOT
    srcp = jnp.concatenate([edge_index[0].astype(i32), loops,
                            jnp.zeros((pad,), i32)])
    dstp = jnp.concatenate([edge_index[1].astype(i32), loops,
                            jnp.full((pad,), N, i32)])

    alpad = jnp.zeros((ACC_R - N, ALW), jnp.float32)

    def sc_pass(zcat, al16):
        al16f = jnp.concatenate([al16, alpad]).reshape(ACC_R * ALW)
        yz, ysf = _sc_edge(zcat.reshape(2 * N, ZW), al16f, srcp, dstp)
        return yz.reshape(2, ACC_R, ZW), ysf.reshape(2, ACC_R, ALW)

    zcat1, al81 = _tc_pre(x, W1, a_src1, a_dst1)
    y1, s1 = sc_pass(zcat1, al81)
    h0 = jnp.zeros((N, C), jnp.float32)
    h1, zcat2, al82 = _tc_mid(y1, s1, b1.reshape(1, C), h0, W2, a_src2, a_dst2)
    y2, s2 = sc_pass(zcat2, al82)
    h2, zcat3, al83 = _tc_mid(y2, s2, b2.reshape(1, C), h1, W3, a_src3, a_dst3)
    y3, s3 = sc_pass(zcat3, al83)
    out = _tc_post(y3, s3, b3.reshape(1, C), h2, lin_w, lin_b.reshape(1, 1))
    return out
